# baseline (speedup 1.0000x reference)
import functools

import jax
import jax.numpy as jnp
from jax.experimental import pallas as pl
from jax.experimental.pallas import tpu as pltpu


def _fused_kernel(a_ref, x_ref, w_ref, bias_ref, o_ref, h_ref, *, bm):
    p = pl.program_id(0)
    i = pl.program_id(1)

    @pl.when(p == 0)
    def _propagate_linear_to_scratch():
        h = jnp.dot(a_ref[...], x_ref[...],
                    preferred_element_type=jnp.float32)
        hw = jax.lax.dot_general(
            h.astype(jnp.bfloat16), w_ref[...], (((1,), (1,)), ((), ())),
            preferred_element_type=jnp.float32,
        )
        h_ref[pl.ds(i * bm, bm), :] = hw.astype(jnp.bfloat16)

    @pl.when(p == 1)
    def _propagate_bias():
        o_ref[...] = jnp.dot(
            a_ref[...], h_ref[...],
            preferred_element_type=jnp.float32,
        ) + bias_ref[...]


def _pick_bm(m):
    for cand in (400, 256, 128, 64, 32, 16, 8):
        if m % cand == 0:
            return cand
    return m


def kernel(x, adj, W, b):
    m, k_total = adj.shape
    f = x.shape[1]
    nh = W.shape[0]
    bm = _pick_bm(m)
    grid = (2, m // bm)
    body = functools.partial(_fused_kernel, bm=bm)
    return pl.pallas_call(
        body,
        grid=grid,
        in_specs=[
            pl.BlockSpec((bm, k_total), lambda p, i: (i, 0)),
            pl.BlockSpec((k_total, f), lambda p, i: (0, 0)),
            pl.BlockSpec((nh, f), lambda p, i: (0, 0)),
            pl.BlockSpec((1, nh), lambda p, i: (0, 0)),
        ],
        out_specs=pl.BlockSpec((bm, nh), lambda p, i: (i * p, 0)),
        out_shape=jax.ShapeDtypeStruct((m, nh), jnp.float32),
        scratch_shapes=[pltpu.VMEM((m, nh), jnp.bfloat16)],
        compiler_params=pltpu.CompilerParams(
            dimension_semantics=("arbitrary", "arbitrary"),
        ),
    )(adj.astype(jnp.bfloat16), x.astype(jnp.bfloat16), W.astype(jnp.bfloat16), b.reshape(1, nh))


# speedup vs baseline: 1.5166x; 1.5166x over previous
import functools

import jax
import jax.numpy as jnp
from jax.experimental import pallas as pl
from jax.experimental.pallas import tpu as pltpu


def _fused_kernel(a_ref, x_ref, w_ref, bias_ref, o_ref, h_ref, *, bm, m_total):
    p = pl.program_id(0)
    i = pl.program_id(1)

    @pl.when(p == 0)
    def _propagate_linear_to_scratch():
        h = jnp.dot(a_ref[...].astype(jnp.bfloat16), x_ref[...],
                    preferred_element_type=jnp.float32)
        hw = jax.lax.dot_general(
            h.astype(jnp.bfloat16), w_ref[...], (((1,), (1,)), ((), ())),
            preferred_element_type=jnp.float32,
        )
        h_ref[pl.ds(i * bm, bm), :] = hw.astype(jnp.bfloat16)

    @pl.when(p == 1)
    def _propagate_bias():
        o_ref[...] = jnp.dot(
            a_ref[...].astype(jnp.bfloat16), h_ref[:m_total, :],
            preferred_element_type=jnp.float32,
        ) + bias_ref[...]


def _pick_bm(m):
    if m >= 512:
        return 512
    return max(8, (m // 8) * 8)


def kernel(x, adj, W, b):
    m, k_total = adj.shape
    f = x.shape[1]
    nh = W.shape[0]
    bm = _pick_bm(m)
    nm = (m + bm - 1) // bm
    grid = (2, nm)
    body = functools.partial(_fused_kernel, bm=bm, m_total=m)
    return pl.pallas_call(
        body,
        grid=grid,
        in_specs=[
            pl.BlockSpec((bm, k_total), lambda p, i: (i, 0)),
            pl.BlockSpec((k_total, f), lambda p, i: (0, 0)),
            pl.BlockSpec((nh, f), lambda p, i: (0, 0)),
            pl.BlockSpec((1, nh), lambda p, i: (0, 0)),
        ],
        out_specs=pl.BlockSpec((bm, nh), lambda p, i: (i * p, 0)),
        out_shape=jax.ShapeDtypeStruct((m, nh), jnp.float32),
        scratch_shapes=[pltpu.VMEM((nm * bm, nh), jnp.bfloat16)],
        compiler_params=pltpu.CompilerParams(
            dimension_semantics=("arbitrary", "arbitrary"),
            vmem_limit_bytes=67108864,
        ),
    )(adj, x.astype(jnp.bfloat16), W.astype(jnp.bfloat16), b.reshape(1, nh))
